# baseline (speedup 1.0000x reference)
import functools

import jax
import jax.numpy as jnp
from jax import lax
from jax.experimental import pallas as pl
from jax.experimental.pallas import tpu as pltpu
from jax.experimental.pallas import tpu_sc as plsc

D = 64
N_WORKERS = 32
SUB = 128
LANES = 16


def _neg_norm(x):
    xs = jnp.maximum(x, jnp.float32(1e-30))
    i = lax.bitcast_convert_type(xs, jnp.int32)
    y = lax.bitcast_convert_type(jnp.int32(0x5F3759DF) - (i >> 1), jnp.float32)
    half = jnp.float32(0.5) * xs
    for _ in range(3):
        y = y * (jnp.float32(1.5) - half * y * y)
    return -(xs * y)


def kernel(heads, rels, tails, times, entity_table, relation_table, time_table):
    B = heads.shape[0]
    rows_per_w = B // N_WORKERS
    n_sub = rows_per_w // SUB

    ent2 = jnp.reshape(entity_table, (-1, 2 * D))
    rel2 = jnp.reshape(relation_table, (-1, 2 * D))
    time2 = jnp.reshape(time_table, (-1, 2 * D))

    mesh = plsc.VectorSubcoreMesh(core_axis_name="c", subcore_axis_name="s")

    @functools.partial(
        pl.kernel,
        mesh=mesh,
        compiler_params=pltpu.CompilerParams(
            needs_layout_passes=False, use_tc_tiling_on_sc=True
        ),
        out_type=jax.ShapeDtypeStruct((B,), jnp.float32),
        scratch_types=[
            pltpu.VMEM((rows_per_w,), jnp.int32),
            pltpu.VMEM((rows_per_w,), jnp.int32),
            pltpu.VMEM((rows_per_w,), jnp.int32),
            pltpu.VMEM((rows_per_w,), jnp.int32),
            pltpu.VMEM((rows_per_w,), jnp.int32),
            pltpu.VMEM((rows_per_w,), jnp.int32),
            pltpu.VMEM((rows_per_w,), jnp.int32),
            pltpu.VMEM((rows_per_w,), jnp.int32),
            pltpu.VMEM((SUB, 2 * D), jnp.float32),
            pltpu.VMEM((SUB, 2 * D), jnp.float32),
            pltpu.VMEM((SUB, 2 * D), jnp.float32),
            pltpu.VMEM((SUB, 2 * D), jnp.float32),
            pltpu.VMEM((rows_per_w,), jnp.float32),
            pltpu.SemaphoreType.DMA,
        ],
    )
    def k(heads_h, rels_h, tails_h, times_h, ent_h, rel_h, time_h, out_h,
          hidx, ridx, tidx, midx, hgi, rgi, tgi, mgi, hb, rb, tb, mb, ob, sem):
        wid = lax.axis_index("s") * 2 + lax.axis_index("c")
        base = wid * rows_per_w

        pltpu.sync_copy(heads_h.at[pl.ds(base, rows_per_w)], hidx)
        pltpu.sync_copy(rels_h.at[pl.ds(base, rows_per_w)], ridx)
        pltpu.sync_copy(tails_h.at[pl.ds(base, rows_per_w)], tidx)
        pltpu.sync_copy(times_h.at[pl.ds(base, rows_per_w)], midx)

        def sbody(i, _):
            sl = pl.ds(i * LANES, LANES)
            hgi[sl] = hidx[sl] >> 1
            rgi[sl] = ridx[sl] >> 1
            tgi[sl] = tidx[sl] >> 1
            mgi[sl] = midx[sl] >> 1
            return _

        lax.fori_loop(0, rows_per_w // LANES, sbody, jnp.int32(0))

        lane = lax.iota(jnp.int32, LANES)

        for j in range(n_sub):
            sl = pl.ds(j * SUB, SUB)
            c1 = pltpu.async_copy(ent_h.at[hgi.at[sl]], hb, sem)
            c2 = pltpu.async_copy(rel_h.at[rgi.at[sl]], rb, sem)
            c3 = pltpu.async_copy(ent_h.at[tgi.at[sl]], tb, sem)
            c4 = pltpu.async_copy(time_h.at[mgi.at[sl]], mb, sem)
            c1.wait()
            c2.wait()
            c3.wait()
            c4.wait()

            def gbody(g, _):
                gsl = pl.ds(j * SUB + g * LANES, LANES)
                hoff = (hidx[gsl] & 1) * D
                roff = (ridx[gsl] & 1) * D
                toff = (tidx[gsl] & 1) * D
                moff = (midx[gsl] & 1) * D
                v = jnp.zeros((LANES,), jnp.float32)
                for r in range(LANES):
                    row = g * LANES + r
                    part = jnp.zeros((LANES,), jnp.float32)
                    for c in range(D // LANES):
                        s = (hb[row, pl.ds(hoff[r] + c * LANES, LANES)]
                             + rb[row, pl.ds(roff[r] + c * LANES, LANES)]
                             + mb[row, pl.ds(moff[r] + c * LANES, LANES)]
                             - tb[row, pl.ds(toff[r] + c * LANES, LANES)])
                        part = part + s * s
                    v = jnp.where(lane == jnp.int32(r), jnp.sum(part), v)
                ob[gsl] = _neg_norm(v)
                return _

            lax.fori_loop(0, SUB // LANES, gbody, jnp.int32(0))

        pltpu.sync_copy(ob, out_h.at[pl.ds(base, rows_per_w)])

    return k(heads, rels, tails, times, ent2, rel2, time2)


# speedup vs baseline: 1.4708x; 1.4708x over previous
import functools

import jax
import jax.numpy as jnp
from jax import lax
from jax.experimental import pallas as pl
from jax.experimental.pallas import tpu as pltpu
from jax.experimental.pallas import tpu_sc as plsc

D = 64
N_WORKERS = 32
LANES = 16
CHUNK = 512
LIST_CAP = 32784
MINI_CAP = 2064
B = 16384
N_ROWS = 2 * B + N_WORKERS

_params = pltpu.CompilerParams(
    needs_layout_passes=False, use_tc_tiling_on_sc=True
)


def _neg_norm(x):
    xs = jnp.maximum(x, jnp.float32(1e-30))
    i = lax.bitcast_convert_type(xs, jnp.int32)
    y = lax.bitcast_convert_type(jnp.int32(0x5F3759DF) - (i >> 1), jnp.float32)
    half = jnp.float32(0.5) * xs
    for _ in range(3):
        y = y * (jnp.float32(1.5) - half * y * y)
    return -(xs * y)


def _gather_rows(heads, tails, et, n_ent):
    mesh = plsc.VectorSubcoreMesh(core_axis_name="c", subcore_axis_name="s")

    @functools.partial(
        pl.kernel,
        mesh=mesh,
        compiler_params=_params,
        out_type=jax.ShapeDtypeStruct((N_ROWS, 2 * D), jnp.float32),
        scratch_types=[
            pltpu.VMEM((2048,), jnp.int32),
            pltpu.VMEM((LIST_CAP,), jnp.int32),
            pltpu.VMEM((LIST_CAP,), jnp.int32),
            pltpu.VMEM((D, CHUNK), jnp.float32),
            pltpu.VMEM((MINI_CAP,), jnp.int32),
            pltpu.VMEM((MINI_CAP,), jnp.int32),
            pltpu.VMEM((128, 2 * D), jnp.float32),
            pltpu.VMEM((1, 128), jnp.int32),
            pltpu.SemaphoreType.DMA,
        ],
    )
    def ka(heads_h, tails_h, et_h, rows_h,
           istage, elist, plist, buf, mini_e, mini_p, rowbuf, idxst, sem):
        w = lax.axis_index("s") * 2 + lax.axis_index("c")
        lane = lax.iota(jnp.int32, LANES)
        dumpv = jnp.full((LANES,), jnp.int32(2 * B)) + w

        n = jnp.int32(0)
        for src, rid_base in ((heads_h, 0), (tails_h, B)):
            for cb in range(B // 2048):
                pltpu.sync_copy(src.at[pl.ds(cb * 2048, 2048)], istage)

                def pbody(v, n, _rb=rid_base + cb * 2048):
                    ev = istage[pl.ds(v * LANES, LANES)]
                    m = (ev >> 15) == w
                    rid = lane + (v * LANES + _rb)
                    plsc.store_compressed(elist.at[pl.ds(n, LANES)], ev, mask=m)
                    plsc.store_compressed(plist.at[pl.ds(n, LANES)], rid, mask=m)
                    return n + plsc.all_reduce_population_count(m)[0]

                n = lax.fori_loop(0, 2048 // LANES, pbody, n)

        for kk in range(8):
            idxst[0, pl.ds(kk * LANES, LANES)] = dumpv

        def flush(_):
            pltpu.async_copy(rowbuf, rows_h.at[idxst.at[0]], sem).wait()
            for kk in range(8):
                idxst[0, pl.ds(kk * LANES, LANES)] = dumpv
            return jnp.int32(0)

        def process_mini(k, f, e0):
            def mbody(mv, f):
                f = lax.cond(f > jnp.int32(112), flush, lambda x: x, f)
                me = mini_e[pl.ds(mv * LANES, LANES)]
                mp = mini_p[pl.ds(mv * LANES, LANES)]
                for r in range(LANES):
                    @pl.when(mv * LANES + r < k)
                    def _():
                        col = jnp.zeros((LANES,), jnp.int32) + (me[r] - e0)
                        fr = f + r
                        for c in range(D // LANES):
                            fv = plsc.load_gather(buf, [lane + c * LANES, col])
                            rowbuf[fr, pl.ds(c * LANES, LANES)] = fv
                        jrow = (fr >> 4) * LANES
                        cur = idxst[0, pl.ds(jrow, LANES)]
                        idxst[0, pl.ds(jrow, LANES)] = jnp.where(
                            lane == (fr & 15), mp[r], cur
                        )
                return f + jnp.minimum(k - mv * LANES, LANES)

            return lax.fori_loop(0, (k + LANES - 1) >> 4, mbody, f)

        def chunk_work(f, e0, e1, clen):
            cps = [
                pltpu.async_copy(
                    et_h.at[pl.ds(a * 8, 8), pl.ds(e0, clen)],
                    buf.at[pl.ds(a * 8, 8), pl.ds(0, clen)],
                    sem,
                )
                for a in range(D // 8)
            ]
            for cp in cps:
                cp.wait()

            nv = (n + LANES - 1) >> 4

            def sbody(v, carry):
                mcnt, f = carry
                ev = elist[pl.ds(v * LANES, LANES)]
                pv = plist[pl.ds(v * LANES, LANES)]
                m = (ev >= e0) & (ev < e1) & ((lane + v * LANES) < n)
                plsc.store_compressed(mini_e.at[pl.ds(mcnt, LANES)], ev, mask=m)
                plsc.store_compressed(mini_p.at[pl.ds(mcnt, LANES)], pv, mask=m)
                mcnt = mcnt + plsc.all_reduce_population_count(m)[0]
                return lax.cond(
                    mcnt >= jnp.int32(MINI_CAP - 32),
                    lambda c, ff: (jnp.int32(0), process_mini(c, ff, e0)),
                    lambda c, ff: (c, ff),
                    mcnt, f,
                )

            mcnt, f = lax.fori_loop(0, nv, sbody, (jnp.int32(0), f))
            return process_mini(mcnt, f, e0)

        e_lo = w << 15
        e_hi = jnp.minimum((w + 1) << 15, n_ent)
        span = jnp.maximum(e_hi - e_lo, 0)
        n_full = span // CHUNK
        rem = span - n_full * CHUNK

        def cbody(ci, f):
            e0 = pl.multiple_of(e_lo + ci * CHUNK, 128)
            return chunk_work(f, e0, e0 + CHUNK, CHUNK)

        f = lax.fori_loop(0, n_full, cbody, jnp.int32(0))
        f = lax.cond(
            rem > 0,
            lambda ff: chunk_work(
                ff, pl.multiple_of(e_lo + n_full * CHUNK, 128), e_hi, 64
            ),
            lambda ff: ff,
            f,
        )
        lax.cond(f > 0, flush, lambda x: x, f)

    return ka(heads, tails, et)


def _score_rows(rels, times, rel2, time2, rows):
    rows_per_w = B // N_WORKERS
    n_sub = rows_per_w // 128
    mesh = plsc.VectorSubcoreMesh(core_axis_name="c", subcore_axis_name="s")

    @functools.partial(
        pl.kernel,
        mesh=mesh,
        compiler_params=_params,
        out_type=jax.ShapeDtypeStruct((B,), jnp.float32),
        scratch_types=[
            pltpu.VMEM((rows_per_w,), jnp.int32),
            pltpu.VMEM((rows_per_w,), jnp.int32),
            pltpu.VMEM((rows_per_w,), jnp.int32),
            pltpu.VMEM((rows_per_w,), jnp.int32),
            pltpu.VMEM((128, 2 * D), jnp.float32),
            pltpu.VMEM((128, 2 * D), jnp.float32),
            pltpu.VMEM((128, 2 * D), jnp.float32),
            pltpu.VMEM((128, 2 * D), jnp.float32),
            pltpu.VMEM((rows_per_w,), jnp.float32),
            pltpu.SemaphoreType.DMA,
        ],
    )
    def kb(rels_h, times_h, rel_h, time_h, rows_h, out_h,
           ridx, midx, rgi, mgi, hb, tb, rb, mb, ob, sem):
        w = lax.axis_index("s") * 2 + lax.axis_index("c")
        base = w * rows_per_w

        pltpu.sync_copy(rels_h.at[pl.ds(base, rows_per_w)], ridx)
        pltpu.sync_copy(times_h.at[pl.ds(base, rows_per_w)], midx)

        def sbody(i, _):
            sl = pl.ds(i * LANES, LANES)
            rgi[sl] = ridx[sl] >> 1
            mgi[sl] = midx[sl] >> 1
            return _

        lax.fori_loop(0, rows_per_w // LANES, sbody, jnp.int32(0))

        lane = lax.iota(jnp.int32, LANES)

        for j in range(n_sub):
            sl = pl.ds(j * 128, 128)
            c1 = pltpu.async_copy(rows_h.at[pl.ds(base + j * 128, 128)], hb, sem)
            c2 = pltpu.async_copy(
                rows_h.at[pl.ds(B + base + j * 128, 128)], tb, sem
            )
            c3 = pltpu.async_copy(rel_h.at[rgi.at[sl]], rb, sem)
            c4 = pltpu.async_copy(time_h.at[mgi.at[sl]], mb, sem)
            c1.wait()
            c2.wait()
            c3.wait()
            c4.wait()

            def gbody(g, _):
                gsl = pl.ds(j * 128 + g * LANES, LANES)
                roff = (ridx[gsl] & 1) * D
                moff = (midx[gsl] & 1) * D
                v = jnp.zeros((LANES,), jnp.float32)
                for r in range(LANES):
                    row = g * LANES + r
                    part = jnp.zeros((LANES,), jnp.float32)
                    for c in range(D // LANES):
                        s = (hb[row, pl.ds(c * LANES, LANES)]
                             + rb[row, pl.ds(roff[r] + c * LANES, LANES)]
                             + mb[row, pl.ds(moff[r] + c * LANES, LANES)]
                             - tb[row, pl.ds(c * LANES, LANES)])
                        part = part + s * s
                    v = jnp.where(lane == jnp.int32(r), jnp.sum(part), v)
                ob[gsl] = _neg_norm(v)
                return _

            lax.fori_loop(0, 128 // LANES, gbody, jnp.int32(0))

        pltpu.sync_copy(ob, out_h.at[pl.ds(base, rows_per_w)])

    return kb(rels, times, rel2, time2, rows)


def kernel(heads, rels, tails, times, entity_table, relation_table, time_table):
    n_ent = entity_table.shape[0]
    et = entity_table.T
    rel2 = jnp.reshape(relation_table, (-1, 2 * D))
    time2 = jnp.reshape(time_table, (-1, 2 * D))
    rows = _gather_rows(heads, tails, et, n_ent)
    return _score_rows(rels, times, rel2, time2, rows)


# speedup vs baseline: 1.9440x; 1.3217x over previous
import functools

import jax
import jax.numpy as jnp
from jax import lax
from jax.experimental import pallas as pl
from jax.experimental.pallas import tpu as pltpu
from jax.experimental.pallas import tpu_sc as plsc

D = 64
N_WORKERS = 32
LANES = 16
CHUNK = 512
LIST_CAP = 32784
MINI_CAP = 2064
B = 16384
N_ROWS = 2 * B + N_WORKERS

_params = pltpu.CompilerParams(
    needs_layout_passes=False, use_tc_tiling_on_sc=True
)


def _neg_norm(x):
    xs = jnp.maximum(x, jnp.float32(1e-30))
    i = lax.bitcast_convert_type(xs, jnp.int32)
    y = lax.bitcast_convert_type(jnp.int32(0x5F3759DF) - (i >> 1), jnp.float32)
    half = jnp.float32(0.5) * xs
    for _ in range(3):
        y = y * (jnp.float32(1.5) - half * y * y)
    return -(xs * y)


def _gather_rows(heads, tails, et, n_ent):
    mesh = plsc.VectorSubcoreMesh(core_axis_name="c", subcore_axis_name="s")

    @functools.partial(
        pl.kernel,
        mesh=mesh,
        compiler_params=_params,
        out_type=jax.ShapeDtypeStruct((N_ROWS, 2 * D), jnp.float32),
        scratch_types=[
            pltpu.VMEM((2048,), jnp.int32),
            pltpu.VMEM((LIST_CAP,), jnp.int32),
            pltpu.VMEM((D, CHUNK), jnp.float32),
            pltpu.VMEM((D, CHUNK), jnp.float32),
            pltpu.VMEM((MINI_CAP,), jnp.int32),
            pltpu.VMEM((128, 2 * D), jnp.float32),
            pltpu.VMEM((1, 128), jnp.int32),
            pltpu.SemaphoreType.DMA,
            pltpu.SemaphoreType.DMA,
            pltpu.SemaphoreType.DMA,
        ],
    )
    def ka(heads_h, tails_h, et_h, rows_h,
           istage, plist, buf0, buf1, mini, rowbuf, idxst, sem0, sem1, semS):
        w = lax.axis_index("s") * 2 + lax.axis_index("c")
        lane = lax.iota(jnp.int32, LANES)
        dumpv = jnp.full((LANES,), jnp.int32(2 * B)) + w

        n = jnp.int32(0)
        for src, rid_base in ((heads_h, 0), (tails_h, B)):
            for cb in range(B // 2048):
                pltpu.sync_copy(src.at[pl.ds(cb * 2048, 2048)], istage)

                def pbody(v, n, _rb=rid_base + cb * 2048):
                    ev = istage[pl.ds(v * LANES, LANES)]
                    m = (ev >> 15) == w
                    rid = lane + (v * LANES + _rb)
                    pk = ((ev & 32767) << 15) | rid
                    plsc.store_compressed(plist.at[pl.ds(n, LANES)], pk, mask=m)
                    return n + plsc.all_reduce_population_count(m)[0]

                n = lax.fori_loop(0, 2048 // LANES, pbody, n)

        plist[pl.ds(n, LANES)] = jnp.full((LANES,), jnp.int32(0x7FFFFFFF))

        for kk in range(8):
            idxst[0, pl.ds(kk * LANES, LANES)] = dumpv

        def flush(_):
            pltpu.async_copy(rowbuf, rows_h.at[idxst.at[0]], semS).wait()
            for kk in range(8):
                idxst[0, pl.ds(kk * LANES, LANES)] = dumpv
            return jnp.int32(0)

        def process_mini(k, f, lo_loc, buf):
            def mbody(mv, f):
                f = lax.cond(f > jnp.int32(112), flush, lambda x: x, f)
                mp = mini[pl.ds(mv * LANES, LANES)]
                for r in range(LANES):
                    @pl.when(mv * LANES + r < k)
                    def _():
                        pk = mp[r]
                        col = jnp.zeros((LANES,), jnp.int32) + (
                            (pk >> 15) - lo_loc
                        )
                        fr = f + r
                        for c in range(D // LANES):
                            fv = plsc.load_gather(buf, [lane + c * LANES, col])
                            rowbuf[fr, pl.ds(c * LANES, LANES)] = fv
                        jrow = (fr >> 4) * LANES
                        cur = idxst[0, pl.ds(jrow, LANES)]
                        idxst[0, pl.ds(jrow, LANES)] = jnp.where(
                            lane == (fr & 15), pk & 32767, cur
                        )
                return f + jnp.minimum(k - mv * LANES, LANES)

            return lax.fori_loop(0, (k + LANES - 1) >> 4, mbody, f)

        nv_ref = [None]

        def chunk_process(f, lo_loc, hi_loc, buf):
            lo = lo_loc << 15
            hi = hi_loc << 15

            def sbody(v, carry):
                mcnt, f = carry
                pk = plist[pl.ds(v * LANES, LANES)]
                m = (pk >= lo) & (pk < hi)
                plsc.store_compressed(mini.at[pl.ds(mcnt, LANES)], pk, mask=m)
                mcnt = mcnt + plsc.all_reduce_population_count(m)[0]
                return lax.cond(
                    mcnt >= jnp.int32(MINI_CAP - 32),
                    lambda c, ff: (jnp.int32(0), process_mini(c, ff, lo_loc, buf)),
                    lambda c, ff: (c, ff),
                    mcnt, f,
                )

            nv = (n + LANES - 1) >> 4
            mcnt, f = lax.fori_loop(0, nv, sbody, (jnp.int32(0), f))
            return process_mini(mcnt, f, lo_loc, buf)

        def issue(e0g, buf, sm):
            for a in range(D // 8):
                pltpu.async_copy(
                    et_h.at[pl.ds(a * 8, 8), pl.ds(e0g, CHUNK)],
                    buf.at[pl.ds(a * 8, 8), :],
                    sm,
                )

        def drain(buf, sm):
            pltpu.make_async_copy(
                et_h.at[pl.ds(0, D), pl.ds(0, CHUNK)], buf, sm
            ).wait()

        e_lo = w << 15
        e_hi = jnp.minimum((w + 1) << 15, n_ent)
        span = jnp.maximum(e_hi - e_lo, 0)
        n_full = span // CHUNK
        rem = span - n_full * CHUNK
        n_pairs = n_full >> 1

        @pl.when(n_full > 0)
        def _():
            issue(pl.multiple_of(e_lo, 128), buf0, sem0)

        def pbody(ci2, f):
            c0 = ci2 * 2
            e0a = pl.multiple_of(e_lo + c0 * CHUNK, 128)
            lo_a = c0 * CHUNK
            issue(pl.multiple_of(e0a + CHUNK, 128), buf1, sem1)
            drain(buf0, sem0)
            f = chunk_process(f, lo_a, lo_a + CHUNK, buf0)

            @pl.when(c0 + 2 < n_full)
            def _():
                issue(pl.multiple_of(e0a + 2 * CHUNK, 128), buf0, sem0)

            drain(buf1, sem1)
            return chunk_process(f, lo_a + CHUNK, lo_a + 2 * CHUNK, buf1)

        f = lax.fori_loop(0, n_pairs, pbody, jnp.int32(0))

        def odd_fn(ff):
            drain(buf0, sem0)
            lo_loc = (n_full - 1) * CHUNK
            return chunk_process(ff, lo_loc, lo_loc + CHUNK, buf0)

        f = lax.cond((n_full & 1) == 1, odd_fn, lambda ff: ff, f)

        def tail_fn(ff):
            e0g = pl.multiple_of(e_lo + n_full * CHUNK, 128)
            cps = [
                pltpu.async_copy(
                    et_h.at[pl.ds(a * 8, 8), pl.ds(e0g, 64)],
                    buf1.at[pl.ds(a * 8, 8), pl.ds(0, 64)],
                    sem1,
                )
                for a in range(D // 8)
            ]
            for cp in cps:
                cp.wait()
            return chunk_process(ff, n_full * CHUNK, span, buf1)

        f = lax.cond(rem > 0, tail_fn, lambda ff: ff, f)
        lax.cond(f > 0, flush, lambda x: x, f)

    return ka(heads, tails, et)


def _score_rows(rels, times, rel2, time2, rows):
    rows_per_w = B // N_WORKERS
    n_sub = rows_per_w // 128
    mesh = plsc.VectorSubcoreMesh(core_axis_name="c", subcore_axis_name="s")

    @functools.partial(
        pl.kernel,
        mesh=mesh,
        compiler_params=_params,
        out_type=jax.ShapeDtypeStruct((B,), jnp.float32),
        scratch_types=[
            pltpu.VMEM((rows_per_w,), jnp.int32),
            pltpu.VMEM((rows_per_w,), jnp.int32),
            pltpu.VMEM((rows_per_w,), jnp.int32),
            pltpu.VMEM((rows_per_w,), jnp.int32),
            pltpu.VMEM((128, 2 * D), jnp.float32),
            pltpu.VMEM((128, 2 * D), jnp.float32),
            pltpu.VMEM((128, 2 * D), jnp.float32),
            pltpu.VMEM((128, 2 * D), jnp.float32),
            pltpu.VMEM((rows_per_w,), jnp.float32),
            pltpu.SemaphoreType.DMA,
        ],
    )
    def kb(rels_h, times_h, rel_h, time_h, rows_h, out_h,
           ridx, midx, rgi, mgi, hb, tb, rb, mb, ob, sem):
        w = lax.axis_index("s") * 2 + lax.axis_index("c")
        base = w * rows_per_w

        pltpu.sync_copy(rels_h.at[pl.ds(base, rows_per_w)], ridx)
        pltpu.sync_copy(times_h.at[pl.ds(base, rows_per_w)], midx)

        def sbody(i, _):
            sl = pl.ds(i * LANES, LANES)
            rgi[sl] = ridx[sl] >> 1
            mgi[sl] = midx[sl] >> 1
            return _

        lax.fori_loop(0, rows_per_w // LANES, sbody, jnp.int32(0))

        lane = lax.iota(jnp.int32, LANES)

        for j in range(n_sub):
            sl = pl.ds(j * 128, 128)
            c1 = pltpu.async_copy(rows_h.at[pl.ds(base + j * 128, 128)], hb, sem)
            c2 = pltpu.async_copy(
                rows_h.at[pl.ds(B + base + j * 128, 128)], tb, sem
            )
            c3 = pltpu.async_copy(rel_h.at[rgi.at[sl]], rb, sem)
            c4 = pltpu.async_copy(time_h.at[mgi.at[sl]], mb, sem)
            c1.wait()
            c2.wait()
            c3.wait()
            c4.wait()

            def gbody(g, _):
                gsl = pl.ds(j * 128 + g * LANES, LANES)
                roff = (ridx[gsl] & 1) * D
                moff = (midx[gsl] & 1) * D
                v = jnp.zeros((LANES,), jnp.float32)
                for r in range(LANES):
                    row = g * LANES + r
                    part = jnp.zeros((LANES,), jnp.float32)
                    for c in range(D // LANES):
                        s = (hb[row, pl.ds(c * LANES, LANES)]
                             + rb[row, pl.ds(roff[r] + c * LANES, LANES)]
                             + mb[row, pl.ds(moff[r] + c * LANES, LANES)]
                             - tb[row, pl.ds(c * LANES, LANES)])
                        part = part + s * s
                    v = jnp.where(lane == jnp.int32(r), jnp.sum(part), v)
                ob[gsl] = _neg_norm(v)
                return _

            lax.fori_loop(0, 128 // LANES, gbody, jnp.int32(0))

        pltpu.sync_copy(ob, out_h.at[pl.ds(base, rows_per_w)])

    return kb(rels, times, rel2, time2, rows)


def kernel(heads, rels, tails, times, entity_table, relation_table, time_table):
    n_ent = entity_table.shape[0]
    et = entity_table.T
    rel2 = jnp.reshape(relation_table, (-1, 2 * D))
    time2 = jnp.reshape(time_table, (-1, 2 * D))
    rows = _gather_rows(heads, tails, et, n_ent)
    return _score_rows(rels, times, rel2, time2, rows)


# speedup vs baseline: 2.4364x; 1.2533x over previous
import functools

import jax
import jax.numpy as jnp
from jax import lax
from jax.experimental import pallas as pl
from jax.experimental.pallas import tpu as pltpu
from jax.experimental.pallas import tpu_sc as plsc

D = 64
N_WORKERS = 32
LANES = 16
CHUNK = 512
LIST_CAP = 32784
MINI_CAP = 2064
B = 16384
N_ROWS = 2 * B + N_WORKERS

_params = pltpu.CompilerParams(
    needs_layout_passes=False, use_tc_tiling_on_sc=True
)


def _neg_norm(x):
    xs = jnp.maximum(x, jnp.float32(1e-30))
    i = lax.bitcast_convert_type(xs, jnp.int32)
    y = lax.bitcast_convert_type(jnp.int32(0x5F3759DF) - (i >> 1), jnp.float32)
    half = jnp.float32(0.5) * xs
    for _ in range(3):
        y = y * (jnp.float32(1.5) - half * y * y)
    return -(xs * y)


def _gather_rows(heads, tails, et, n_ent):
    mesh = plsc.VectorSubcoreMesh(core_axis_name="c", subcore_axis_name="s")

    @functools.partial(
        pl.kernel,
        mesh=mesh,
        compiler_params=_params,
        out_type=jax.ShapeDtypeStruct((N_ROWS, 2 * D), jnp.float32),
        scratch_types=[
            pltpu.VMEM((2048,), jnp.int32),
            pltpu.VMEM((LIST_CAP,), jnp.int32),
            pltpu.VMEM((D, CHUNK), jnp.float32),
            pltpu.VMEM((D, CHUNK), jnp.float32),
            pltpu.VMEM((MINI_CAP,), jnp.int32),
            pltpu.VMEM((128, 2 * D), jnp.float32),
            pltpu.VMEM((1, 128), jnp.int32),
            pltpu.SemaphoreType.DMA,
            pltpu.SemaphoreType.DMA,
            pltpu.SemaphoreType.DMA,
        ],
    )
    def ka(heads_h, tails_h, et_h, rows_h,
           istage, plist, buf0, buf1, mini, rowbuf, idxst, sem0, sem1, semS):
        w = lax.axis_index("s") * 2 + lax.axis_index("c")
        lane = lax.iota(jnp.int32, LANES)
        dumpv = jnp.full((LANES,), jnp.int32(2 * B)) + w

        n = jnp.int32(0)
        for src, rid_base in ((heads_h, 0), (tails_h, B)):
            for cb in range(B // 2048):
                pltpu.sync_copy(src.at[pl.ds(cb * 2048, 2048)], istage)

                def pbody(v, n, _rb=rid_base + cb * 2048):
                    ev = istage[pl.ds(v * LANES, LANES)]
                    m = (ev >> 15) == w
                    rid = lane + (v * LANES + _rb)
                    pk = ((ev & 32767) << 15) | rid
                    plsc.store_compressed(plist.at[pl.ds(n, LANES)], pk, mask=m)
                    return n + plsc.all_reduce_population_count(m)[0]

                n = lax.fori_loop(0, 2048 // LANES, pbody, n)

        plist[pl.ds(n, LANES)] = jnp.full((LANES,), jnp.int32(0x7FFFFFFF))

        for kk in range(8):
            idxst[0, pl.ds(kk * LANES, LANES)] = dumpv

        def flush(_):
            pltpu.async_copy(rowbuf, rows_h.at[idxst.at[0]], semS).wait()
            for kk in range(8):
                idxst[0, pl.ds(kk * LANES, LANES)] = dumpv
            return jnp.int32(0)

        def process_mini(k, f, lo_loc, buf):
            def mbody(mv, f):
                f = lax.cond(f > jnp.int32(112), flush, lambda x: x, f)
                mp = mini[pl.ds(mv * LANES, LANES)]
                for r in range(LANES):
                    @pl.when(mv * LANES + r < k)
                    def _():
                        pk = mp[r]
                        col = jnp.zeros((LANES,), jnp.int32) + (
                            (pk >> 15) - lo_loc
                        )
                        fr = f + r
                        for c in range(D // LANES):
                            fv = plsc.load_gather(buf, [lane + c * LANES, col])
                            rowbuf[fr, pl.ds(c * LANES, LANES)] = fv
                        jrow = (fr >> 4) * LANES
                        cur = idxst[0, pl.ds(jrow, LANES)]
                        idxst[0, pl.ds(jrow, LANES)] = jnp.where(
                            lane == (fr & 15), pk & 32767, cur
                        )
                return f + jnp.minimum(k - mv * LANES, LANES)

            return lax.fori_loop(0, (k + LANES - 1) >> 4, mbody, f)

        def chunk_process(f, lo_loc, hi_loc, buf):
            lo = lo_loc << 15
            hi = hi_loc << 15
            nv = (n + LANES - 1) >> 4
            nb = (nv + 63) >> 6

            def sbody(v, mcnt):
                pk = plist[pl.ds(v * LANES, LANES)]
                m = (pk >= lo) & (pk < hi)
                plsc.store_compressed(mini.at[pl.ds(mcnt, LANES)], pk, mask=m)
                return mcnt + plsc.all_reduce_population_count(m)[0]

            def bbody(bi, carry):
                mcnt, f = carry
                mcnt, f = lax.cond(
                    mcnt >= jnp.int32(MINI_CAP - 1040),
                    lambda c, ff: (jnp.int32(0), process_mini(c, ff, lo_loc, buf)),
                    lambda c, ff: (c, ff),
                    mcnt, f,
                )
                v0 = bi * 64
                mcnt = lax.fori_loop(
                    v0, jnp.minimum(v0 + 64, nv), sbody, mcnt
                )
                return (mcnt, f)

            mcnt, f = lax.fori_loop(0, nb, bbody, (jnp.int32(0), f))
            return process_mini(mcnt, f, lo_loc, buf)

        def issue(e0g, buf, sm):
            pltpu.async_copy(
                et_h.at[pl.ds(0, D), pl.ds(e0g, CHUNK)], buf, sm
            )

        def drain(buf, sm):
            pltpu.make_async_copy(
                et_h.at[pl.ds(0, D), pl.ds(0, CHUNK)], buf, sm
            ).wait()

        e_lo = w << 15
        e_hi = jnp.minimum((w + 1) << 15, n_ent)
        span = jnp.maximum(e_hi - e_lo, 0)
        n_full = span // CHUNK
        rem = span - n_full * CHUNK
        n_pairs = n_full >> 1

        @pl.when(n_full > 0)
        def _():
            issue(pl.multiple_of(e_lo, 128), buf0, sem0)

        def pbody(ci2, f):
            c0 = ci2 * 2
            e0a = pl.multiple_of(e_lo + c0 * CHUNK, 128)
            lo_a = c0 * CHUNK
            issue(pl.multiple_of(e0a + CHUNK, 128), buf1, sem1)
            drain(buf0, sem0)
            f = chunk_process(f, lo_a, lo_a + CHUNK, buf0)

            @pl.when(c0 + 2 < n_full)
            def _():
                issue(pl.multiple_of(e0a + 2 * CHUNK, 128), buf0, sem0)

            drain(buf1, sem1)
            return chunk_process(f, lo_a + CHUNK, lo_a + 2 * CHUNK, buf1)

        f = lax.fori_loop(0, n_pairs, pbody, jnp.int32(0))

        def odd_fn(ff):
            drain(buf0, sem0)
            lo_loc = (n_full - 1) * CHUNK
            return chunk_process(ff, lo_loc, lo_loc + CHUNK, buf0)

        f = lax.cond((n_full & 1) == 1, odd_fn, lambda ff: ff, f)

        def tail_fn(ff):
            e0g = pl.multiple_of(e_lo + n_full * CHUNK, 128)
            cps = [
                pltpu.async_copy(
                    et_h.at[pl.ds(a * 8, 8), pl.ds(e0g, 64)],
                    buf1.at[pl.ds(a * 8, 8), pl.ds(0, 64)],
                    sem1,
                )
                for a in range(D // 8)
            ]
            for cp in cps:
                cp.wait()
            return chunk_process(ff, n_full * CHUNK, span, buf1)

        f = lax.cond(rem > 0, tail_fn, lambda ff: ff, f)
        lax.cond(f > 0, flush, lambda x: x, f)

    return ka(heads, tails, et)


def _score_rows(rels, times, rel2, time2, rows):
    rows_per_w = B // N_WORKERS
    n_sub = rows_per_w // 128
    mesh = plsc.VectorSubcoreMesh(core_axis_name="c", subcore_axis_name="s")

    @functools.partial(
        pl.kernel,
        mesh=mesh,
        compiler_params=_params,
        out_type=jax.ShapeDtypeStruct((B,), jnp.float32),
        scratch_types=[
            pltpu.VMEM((rows_per_w,), jnp.int32),
            pltpu.VMEM((rows_per_w,), jnp.int32),
            pltpu.VMEM((rows_per_w,), jnp.int32),
            pltpu.VMEM((rows_per_w,), jnp.int32),
            pltpu.VMEM((128, 2 * D), jnp.float32),
            pltpu.VMEM((128, 2 * D), jnp.float32),
            pltpu.VMEM((128, 2 * D), jnp.float32),
            pltpu.VMEM((128, 2 * D), jnp.float32),
            pltpu.VMEM((rows_per_w,), jnp.float32),
            pltpu.SemaphoreType.DMA,
        ],
    )
    def kb(rels_h, times_h, rel_h, time_h, rows_h, out_h,
           ridx, midx, rgi, mgi, hb, tb, rb, mb, ob, sem):
        w = lax.axis_index("s") * 2 + lax.axis_index("c")
        base = w * rows_per_w

        pltpu.sync_copy(rels_h.at[pl.ds(base, rows_per_w)], ridx)
        pltpu.sync_copy(times_h.at[pl.ds(base, rows_per_w)], midx)

        def sbody(i, _):
            sl = pl.ds(i * LANES, LANES)
            rgi[sl] = ridx[sl] >> 1
            mgi[sl] = midx[sl] >> 1
            return _

        lax.fori_loop(0, rows_per_w // LANES, sbody, jnp.int32(0))

        lane = lax.iota(jnp.int32, LANES)

        for j in range(n_sub):
            sl = pl.ds(j * 128, 128)
            c1 = pltpu.async_copy(rows_h.at[pl.ds(base + j * 128, 128)], hb, sem)
            c2 = pltpu.async_copy(
                rows_h.at[pl.ds(B + base + j * 128, 128)], tb, sem
            )
            c3 = pltpu.async_copy(rel_h.at[rgi.at[sl]], rb, sem)
            c4 = pltpu.async_copy(time_h.at[mgi.at[sl]], mb, sem)
            c1.wait()
            c2.wait()
            c3.wait()
            c4.wait()

            def gbody(g, _):
                gsl = pl.ds(j * 128 + g * LANES, LANES)
                roff = (ridx[gsl] & 1) * D
                moff = (midx[gsl] & 1) * D
                v = jnp.zeros((LANES,), jnp.float32)
                for r in range(LANES):
                    row = g * LANES + r
                    part = jnp.zeros((LANES,), jnp.float32)
                    for c in range(D // LANES):
                        s = (hb[row, pl.ds(c * LANES, LANES)]
                             + rb[row, pl.ds(roff[r] + c * LANES, LANES)]
                             + mb[row, pl.ds(moff[r] + c * LANES, LANES)]
                             - tb[row, pl.ds(c * LANES, LANES)])
                        part = part + s * s
                    v = jnp.where(lane == jnp.int32(r), jnp.sum(part), v)
                ob[gsl] = _neg_norm(v)
                return _

            lax.fori_loop(0, 128 // LANES, gbody, jnp.int32(0))

        pltpu.sync_copy(ob, out_h.at[pl.ds(base, rows_per_w)])

    return kb(rels, times, rel2, time2, rows)


def kernel(heads, rels, tails, times, entity_table, relation_table, time_table):
    n_ent = entity_table.shape[0]
    et = entity_table.T
    rel2 = jnp.reshape(relation_table, (-1, 2 * D))
    time2 = jnp.reshape(time_table, (-1, 2 * D))
    rows = _gather_rows(heads, tails, et, n_ent)
    return _score_rows(rels, times, rel2, time2, rows)


# speedup vs baseline: 2.7775x; 1.1400x over previous
import functools

import jax
import jax.numpy as jnp
from jax import lax
from jax.experimental import pallas as pl
from jax.experimental.pallas import tpu as pltpu
from jax.experimental.pallas import tpu_sc as plsc

D = 64
N_WORKERS = 32
LANES = 16
CHUNK = 512
LIST_CAP = 32784
MINI_CAP = 2064
GCAP = 640
B = 16384
N_ROWS = 2 * B + N_WORKERS

_params = pltpu.CompilerParams(
    needs_layout_passes=False, use_tc_tiling_on_sc=True
)


def _neg_norm(x):
    xs = jnp.maximum(x, jnp.float32(1e-30))
    i = lax.bitcast_convert_type(xs, jnp.int32)
    y = lax.bitcast_convert_type(jnp.int32(0x5F3759DF) - (i >> 1), jnp.float32)
    half = jnp.float32(0.5) * xs
    for _ in range(3):
        y = y * (jnp.float32(1.5) - half * y * y)
    return -(xs * y)


def _gather_rows(heads, tails, et, n_ent):
    mesh = plsc.VectorSubcoreMesh(core_axis_name="c", subcore_axis_name="s")

    @functools.partial(
        pl.kernel,
        mesh=mesh,
        compiler_params=_params,
        out_type=jax.ShapeDtypeStruct((N_ROWS, 2 * D), jnp.float32),
        scratch_types=[
            pltpu.VMEM((2048,), jnp.int32),
            pltpu.VMEM((LIST_CAP,), jnp.int32),
            pltpu.VMEM((8 * GCAP,), jnp.int32),
            pltpu.VMEM((D, CHUNK), jnp.float32),
            pltpu.VMEM((D, CHUNK), jnp.float32),
            pltpu.VMEM((MINI_CAP,), jnp.int32),
            pltpu.VMEM((128, 2 * D), jnp.float32),
            pltpu.VMEM((1, 128), jnp.int32),
            pltpu.SemaphoreType.DMA,
            pltpu.SemaphoreType.DMA,
            pltpu.SemaphoreType.DMA,
        ],
    )
    def ka(heads_h, tails_h, et_h, rows_h,
           istage, plist, glist, buf0, buf1, mini, rowbuf, idxst,
           sem0, sem1, semS):
        w = lax.axis_index("s") * 2 + lax.axis_index("c")
        lane = lax.iota(jnp.int32, LANES)
        dumpv = jnp.full((LANES,), jnp.int32(2 * B)) + w

        n = jnp.int32(0)
        for src, rid_base in ((heads_h, 0), (tails_h, B)):
            for cb in range(B // 2048):
                pltpu.sync_copy(src.at[pl.ds(cb * 2048, 2048)], istage)

                def pbody(v, n, _rb=rid_base + cb * 2048):
                    ev = istage[pl.ds(v * LANES, LANES)]
                    m = (ev >> 15) == w
                    rid = lane + (v * LANES + _rb)
                    pk = ((ev & 32767) << 15) | rid
                    plsc.store_compressed(plist.at[pl.ds(n, LANES)], pk, mask=m)
                    return n + plsc.all_reduce_population_count(m)[0]

                n = lax.fori_loop(0, 2048 // LANES, pbody, n)

        sent = jnp.full((LANES,), jnp.int32(0x7FFFFFFF))
        plist[pl.ds(n, LANES)] = sent

        nv_all = (n + LANES - 1) >> 4

        def gbody(v, carry):
            pk = plist[pl.ds(v * LANES, LANES)]
            gid = pk >> 27
            out = []
            for g in range(8):
                m = gid == g
                pos = jnp.minimum(carry[g], GCAP - LANES)
                plsc.store_compressed(
                    glist.at[pl.ds(g * GCAP + pos, LANES)], pk, mask=m
                )
                out.append(carry[g] + plsc.all_reduce_population_count(m)[0])
            return tuple(out)

        gcnts = lax.fori_loop(
            0, nv_all, gbody, (jnp.int32(0),) * 8
        )
        gvec = jnp.zeros((LANES,), jnp.int32)
        for g in range(8):
            glist[pl.ds(g * GCAP + jnp.minimum(gcnts[g], GCAP - LANES), LANES)] = sent
            gvec = jnp.where(lane == g, gcnts[g], gvec)

        for kk in range(8):
            idxst[0, pl.ds(kk * LANES, LANES)] = dumpv

        def flush(_):
            pltpu.async_copy(rowbuf, rows_h.at[idxst.at[0]], semS).wait()
            for kk in range(8):
                idxst[0, pl.ds(kk * LANES, LANES)] = dumpv
            return jnp.int32(0)

        def process_mini(k, f, lo_loc, buf):
            def mbody(mv, f):
                f = lax.cond(f > jnp.int32(112), flush, lambda x: x, f)
                mp = mini[pl.ds(mv * LANES, LANES)]
                for r in range(LANES):
                    @pl.when(mv * LANES + r < k)
                    def _():
                        pk = mp[r]
                        col = jnp.zeros((LANES,), jnp.int32) + (
                            (pk >> 15) - lo_loc
                        )
                        fr = f + r
                        for c in range(D // LANES):
                            fv = plsc.load_gather(buf, [lane + c * LANES, col])
                            rowbuf[fr, pl.ds(c * LANES, LANES)] = fv
                        jrow = (fr >> 4) * LANES
                        cur = idxst[0, pl.ds(jrow, LANES)]
                        idxst[0, pl.ds(jrow, LANES)] = jnp.where(
                            lane == (fr & 15), pk & 32767, cur
                        )
                return f + jnp.minimum(k - mv * LANES, LANES)

            return lax.fori_loop(0, (k + LANES - 1) >> 4, mbody, f)

        def chunk_process(f, lo_loc, hi_loc, buf):
            lo = lo_loc << 15
            hi = hi_loc << 15
            g = lo_loc >> 12
            cnt_g = jnp.sum(jnp.where(lane == g, gvec, jnp.int32(0)))

            def scan_vreg(ref):
                def sbody(v, mcnt):
                    pk = ref[pl.ds(v * LANES, LANES)]
                    m = (pk >= lo) & (pk < hi)
                    plsc.store_compressed(
                        mini.at[pl.ds(mcnt, LANES)], pk, mask=m
                    )
                    return mcnt + plsc.all_reduce_population_count(m)[0]

                return sbody

            def group_scan(ff):
                v0 = g * (GCAP // LANES)
                mcnt = lax.fori_loop(
                    v0, v0 + ((cnt_g + LANES - 1) >> 4), scan_vreg(glist),
                    jnp.int32(0),
                )
                return (mcnt, ff)

            def full_scan(ff):
                nv = (n + LANES - 1) >> 4
                nb = (nv + 63) >> 6

                def bbody(bi, carry):
                    mcnt, ff = carry
                    mcnt, ff = lax.cond(
                        mcnt >= jnp.int32(MINI_CAP - 1040),
                        lambda c, f2: (
                            jnp.int32(0), process_mini(c, f2, lo_loc, buf)
                        ),
                        lambda c, f2: (c, f2),
                        mcnt, ff,
                    )
                    v0 = bi * 64
                    mcnt = lax.fori_loop(
                        v0, jnp.minimum(v0 + 64, nv), scan_vreg(plist), mcnt
                    )
                    return (mcnt, ff)

                return lax.fori_loop(0, nb, bbody, (jnp.int32(0), ff))

            mcnt, f = lax.cond(
                cnt_g > jnp.int32(GCAP - LANES), full_scan, group_scan, f
            )
            return process_mini(mcnt, f, lo_loc, buf)

        def issue(e0g, buf, sm):
            pltpu.async_copy(
                et_h.at[pl.ds(0, D), pl.ds(e0g, CHUNK)], buf, sm
            )

        def drain(buf, sm):
            pltpu.make_async_copy(
                et_h.at[pl.ds(0, D), pl.ds(0, CHUNK)], buf, sm
            ).wait()

        e_lo = w << 15
        e_hi = jnp.minimum((w + 1) << 15, n_ent)
        span = jnp.maximum(e_hi - e_lo, 0)
        n_full = span // CHUNK
        rem = span - n_full * CHUNK
        n_pairs = n_full >> 1

        @pl.when(n_full > 0)
        def _():
            issue(pl.multiple_of(e_lo, 128), buf0, sem0)

        def pbody(ci2, f):
            c0 = ci2 * 2
            e0a = pl.multiple_of(e_lo + c0 * CHUNK, 128)
            lo_a = c0 * CHUNK
            issue(pl.multiple_of(e0a + CHUNK, 128), buf1, sem1)
            drain(buf0, sem0)
            f = chunk_process(f, lo_a, lo_a + CHUNK, buf0)

            @pl.when(c0 + 2 < n_full)
            def _():
                issue(pl.multiple_of(e0a + 2 * CHUNK, 128), buf0, sem0)

            drain(buf1, sem1)
            return chunk_process(f, lo_a + CHUNK, lo_a + 2 * CHUNK, buf1)

        f = lax.fori_loop(0, n_pairs, pbody, jnp.int32(0))

        def odd_fn(ff):
            drain(buf0, sem0)
            lo_loc = (n_full - 1) * CHUNK
            return chunk_process(ff, lo_loc, lo_loc + CHUNK, buf0)

        f = lax.cond((n_full & 1) == 1, odd_fn, lambda ff: ff, f)

        def tail_fn(ff):
            e0g = pl.multiple_of(e_lo + n_full * CHUNK, 128)
            cps = [
                pltpu.async_copy(
                    et_h.at[pl.ds(a * 8, 8), pl.ds(e0g, 64)],
                    buf1.at[pl.ds(a * 8, 8), pl.ds(0, 64)],
                    sem1,
                )
                for a in range(D // 8)
            ]
            for cp in cps:
                cp.wait()
            return chunk_process(ff, n_full * CHUNK, span, buf1)

        f = lax.cond(rem > 0, tail_fn, lambda ff: ff, f)
        lax.cond(f > 0, flush, lambda x: x, f)

    return ka(heads, tails, et)


def _score_rows(rels, times, rel2, time2, rows):
    rows_per_w = B // N_WORKERS
    n_sub = rows_per_w // 128
    mesh = plsc.VectorSubcoreMesh(core_axis_name="c", subcore_axis_name="s")

    @functools.partial(
        pl.kernel,
        mesh=mesh,
        compiler_params=_params,
        out_type=jax.ShapeDtypeStruct((B,), jnp.float32),
        scratch_types=[
            pltpu.VMEM((rows_per_w,), jnp.int32),
            pltpu.VMEM((rows_per_w,), jnp.int32),
            pltpu.VMEM((rows_per_w,), jnp.int32),
            pltpu.VMEM((rows_per_w,), jnp.int32),
            pltpu.VMEM((128, 2 * D), jnp.float32),
            pltpu.VMEM((128, 2 * D), jnp.float32),
            pltpu.VMEM((128, 2 * D), jnp.float32),
            pltpu.VMEM((128, 2 * D), jnp.float32),
            pltpu.VMEM((rows_per_w,), jnp.float32),
            pltpu.SemaphoreType.DMA,
        ],
    )
    def kb(rels_h, times_h, rel_h, time_h, rows_h, out_h,
           ridx, midx, rgi, mgi, hb, tb, rb, mb, ob, sem):
        w = lax.axis_index("s") * 2 + lax.axis_index("c")
        base = w * rows_per_w

        pltpu.sync_copy(rels_h.at[pl.ds(base, rows_per_w)], ridx)
        pltpu.sync_copy(times_h.at[pl.ds(base, rows_per_w)], midx)

        def sbody(i, _):
            sl = pl.ds(i * LANES, LANES)
            rgi[sl] = ridx[sl] >> 1
            mgi[sl] = midx[sl] >> 1
            return _

        lax.fori_loop(0, rows_per_w // LANES, sbody, jnp.int32(0))

        lane = lax.iota(jnp.int32, LANES)

        for j in range(n_sub):
            sl = pl.ds(j * 128, 128)
            c1 = pltpu.async_copy(rows_h.at[pl.ds(base + j * 128, 128)], hb, sem)
            c2 = pltpu.async_copy(
                rows_h.at[pl.ds(B + base + j * 128, 128)], tb, sem
            )
            c3 = pltpu.async_copy(rel_h.at[rgi.at[sl]], rb, sem)
            c4 = pltpu.async_copy(time_h.at[mgi.at[sl]], mb, sem)
            c1.wait()
            c2.wait()
            c3.wait()
            c4.wait()

            def gbody(g, _):
                gsl = pl.ds(j * 128 + g * LANES, LANES)
                roff = (ridx[gsl] & 1) * D
                moff = (midx[gsl] & 1) * D
                v = jnp.zeros((LANES,), jnp.float32)
                for r in range(LANES):
                    row = g * LANES + r
                    part = jnp.zeros((LANES,), jnp.float32)
                    for c in range(D // LANES):
                        s = (hb[row, pl.ds(c * LANES, LANES)]
                             + rb[row, pl.ds(roff[r] + c * LANES, LANES)]
                             + mb[row, pl.ds(moff[r] + c * LANES, LANES)]
                             - tb[row, pl.ds(c * LANES, LANES)])
                        part = part + s * s
                    v = jnp.where(lane == jnp.int32(r), jnp.sum(part), v)
                ob[gsl] = _neg_norm(v)
                return _

            lax.fori_loop(0, 128 // LANES, gbody, jnp.int32(0))

        pltpu.sync_copy(ob, out_h.at[pl.ds(base, rows_per_w)])

    return kb(rels, times, rel2, time2, rows)


def kernel(heads, rels, tails, times, entity_table, relation_table, time_table):
    n_ent = entity_table.shape[0]
    et = entity_table.T
    rel2 = jnp.reshape(relation_table, (-1, 2 * D))
    time2 = jnp.reshape(time_table, (-1, 2 * D))
    rows = _gather_rows(heads, tails, et, n_ent)
    return _score_rows(rels, times, rel2, time2, rows)


# speedup vs baseline: 2.7865x; 1.0032x over previous
import functools

import jax
import jax.numpy as jnp
from jax import lax
from jax.experimental import pallas as pl
from jax.experimental.pallas import tpu as pltpu
from jax.experimental.pallas import tpu_sc as plsc

D = 64
N_WORKERS = 32
LANES = 16
CHUNK = 512
LIST_CAP = 32784
MINI_CAP = 2064
GCAP = 640
B = 16384
N_ROWS = 2 * B + N_WORKERS

_params = pltpu.CompilerParams(
    needs_layout_passes=False, use_tc_tiling_on_sc=True
)


def _neg_norm(x):
    xs = jnp.maximum(x, jnp.float32(1e-30))
    i = lax.bitcast_convert_type(xs, jnp.int32)
    y = lax.bitcast_convert_type(jnp.int32(0x5F3759DF) - (i >> 1), jnp.float32)
    half = jnp.float32(0.5) * xs
    for _ in range(3):
        y = y * (jnp.float32(1.5) - half * y * y)
    return -(xs * y)


def _gather_rows(heads, tails, et, n_ent):
    mesh = plsc.VectorSubcoreMesh(core_axis_name="c", subcore_axis_name="s")

    @functools.partial(
        pl.kernel,
        mesh=mesh,
        compiler_params=_params,
        out_type=jax.ShapeDtypeStruct((N_ROWS, 2 * D), jnp.float32),
        scratch_types=[
            pltpu.VMEM((2048,), jnp.int32),
            pltpu.VMEM((LIST_CAP,), jnp.int32),
            pltpu.VMEM((8 * GCAP,), jnp.int32),
            pltpu.VMEM((D, CHUNK), jnp.float32),
            pltpu.VMEM((D, CHUNK), jnp.float32),
            pltpu.VMEM((MINI_CAP,), jnp.int32),
            pltpu.VMEM((128, 2 * D), jnp.float32),
            pltpu.VMEM((1, 128), jnp.int32),
            pltpu.SemaphoreType.DMA,
            pltpu.SemaphoreType.DMA,
            pltpu.SemaphoreType.DMA,
        ],
    )
    def ka(heads_h, tails_h, et_h, rows_h,
           istage, plist, glist, buf0, buf1, mini, rowbuf, idxst,
           sem0, sem1, semS):
        w = lax.axis_index("s") * 2 + lax.axis_index("c")
        lane = lax.iota(jnp.int32, LANES)
        dumpv = jnp.full((LANES,), jnp.int32(2 * B)) + w

        e_lo = w << 15
        e_hi = jnp.minimum((w + 1) << 15, n_ent)
        span = jnp.maximum(e_hi - e_lo, 0)
        n_full = span // CHUNK
        rem = span - n_full * CHUNK

        def issue(e0g, buf, sm):
            pltpu.async_copy(
                et_h.at[pl.ds(0, D), pl.ds(e0g, CHUNK)], buf, sm
            )

        @pl.when(n_full > 0)
        def _():
            issue(pl.multiple_of(e_lo, 128), buf0, sem0)

        @pl.when(n_full > 1)
        def _():
            issue(pl.multiple_of(e_lo + CHUNK, 128), buf1, sem1)

        n = jnp.int32(0)
        for src, rid_base in ((heads_h, 0), (tails_h, B)):
            for cb in range(B // 2048):
                pltpu.sync_copy(src.at[pl.ds(cb * 2048, 2048)], istage)

                def pbody(v4, n, _rb=rid_base + cb * 2048):
                    for u in range(4):
                        v = v4 * 4 + u
                        ev = istage[pl.ds(v * LANES, LANES)]
                        m = (ev >> 15) == w
                        rid = lane + (v * LANES + _rb)
                        pk = ((ev & 32767) << 15) | rid
                        plsc.store_compressed(
                            plist.at[pl.ds(n, LANES)], pk, mask=m
                        )
                        n = n + plsc.all_reduce_population_count(m)[0]
                    return n

                n = lax.fori_loop(0, 2048 // LANES // 4, pbody, n)

        sent = jnp.full((LANES,), jnp.int32(0x7FFFFFFF))
        plist[pl.ds(n, LANES)] = sent

        nv_all = (n + LANES - 1) >> 4

        def gbody(v, carry):
            pk = plist[pl.ds(v * LANES, LANES)]
            gid = pk >> 27
            out = []
            for g in range(8):
                m = gid == g
                pos = jnp.minimum(carry[g], GCAP - LANES)
                plsc.store_compressed(
                    glist.at[pl.ds(g * GCAP + pos, LANES)], pk, mask=m
                )
                out.append(carry[g] + plsc.all_reduce_population_count(m)[0])
            return tuple(out)

        gcnts = lax.fori_loop(
            0, nv_all, gbody, (jnp.int32(0),) * 8
        )
        gvec = jnp.zeros((LANES,), jnp.int32)
        for g in range(8):
            glist[pl.ds(g * GCAP + jnp.minimum(gcnts[g], GCAP - LANES), LANES)] = sent
            gvec = jnp.where(lane == g, gcnts[g], gvec)

        for kk in range(8):
            idxst[0, pl.ds(kk * LANES, LANES)] = dumpv

        def flush(_):
            pltpu.async_copy(rowbuf, rows_h.at[idxst.at[0]], semS).wait()
            for kk in range(8):
                idxst[0, pl.ds(kk * LANES, LANES)] = dumpv
            return jnp.int32(0)

        def process_mini(k, f, lo_loc, buf):
            def mbody(mv, f):
                f = lax.cond(f > jnp.int32(112), flush, lambda x: x, f)
                mp = mini[pl.ds(mv * LANES, LANES)]
                for r in range(LANES):
                    @pl.when(mv * LANES + r < k)
                    def _():
                        pk = mp[r]
                        col = jnp.zeros((LANES,), jnp.int32) + (
                            (pk >> 15) - lo_loc
                        )
                        fr = f + r
                        for c in range(D // LANES):
                            fv = plsc.load_gather(buf, [lane + c * LANES, col])
                            rowbuf[fr, pl.ds(c * LANES, LANES)] = fv
                        jrow = (fr >> 4) * LANES
                        cur = idxst[0, pl.ds(jrow, LANES)]
                        idxst[0, pl.ds(jrow, LANES)] = jnp.where(
                            lane == (fr & 15), pk & 32767, cur
                        )
                return f + jnp.minimum(k - mv * LANES, LANES)

            return lax.fori_loop(0, (k + LANES - 1) >> 4, mbody, f)

        def chunk_process(f, lo_loc, hi_loc, buf):
            lo = lo_loc << 15
            hi = hi_loc << 15
            g = lo_loc >> 12
            cnt_g = jnp.sum(jnp.where(lane == g, gvec, jnp.int32(0)))

            def scan_vreg(ref):
                def sbody(v, mcnt):
                    pk = ref[pl.ds(v * LANES, LANES)]
                    m = (pk >= lo) & (pk < hi)
                    plsc.store_compressed(
                        mini.at[pl.ds(mcnt, LANES)], pk, mask=m
                    )
                    return mcnt + plsc.all_reduce_population_count(m)[0]

                return sbody

            def group_scan(ff):
                v0 = g * (GCAP // LANES)
                mcnt = lax.fori_loop(
                    v0, v0 + ((cnt_g + LANES - 1) >> 4), scan_vreg(glist),
                    jnp.int32(0),
                )
                return (mcnt, ff)

            def full_scan(ff):
                nv = (n + LANES - 1) >> 4
                nb = (nv + 63) >> 6

                def bbody(bi, carry):
                    mcnt, ff = carry
                    mcnt, ff = lax.cond(
                        mcnt >= jnp.int32(MINI_CAP - 1040),
                        lambda c, f2: (
                            jnp.int32(0), process_mini(c, f2, lo_loc, buf)
                        ),
                        lambda c, f2: (c, f2),
                        mcnt, ff,
                    )
                    v0 = bi * 64
                    mcnt = lax.fori_loop(
                        v0, jnp.minimum(v0 + 64, nv), scan_vreg(plist), mcnt
                    )
                    return (mcnt, ff)

                return lax.fori_loop(0, nb, bbody, (jnp.int32(0), ff))

            mcnt, f = lax.cond(
                cnt_g > jnp.int32(GCAP - LANES), full_scan, group_scan, f
            )
            return process_mini(mcnt, f, lo_loc, buf)

        def drain(buf, sm):
            pltpu.make_async_copy(
                et_h.at[pl.ds(0, D), pl.ds(0, CHUNK)], buf, sm
            ).wait()

        n_pairs = n_full >> 1

        def pbody(ci2, f):
            c0 = ci2 * 2
            e0a = pl.multiple_of(e_lo + c0 * CHUNK, 128)
            lo_a = c0 * CHUNK
            drain(buf0, sem0)
            f = chunk_process(f, lo_a, lo_a + CHUNK, buf0)

            @pl.when(c0 + 2 < n_full)
            def _():
                issue(pl.multiple_of(e0a + 2 * CHUNK, 128), buf0, sem0)

            drain(buf1, sem1)
            f = chunk_process(f, lo_a + CHUNK, lo_a + 2 * CHUNK, buf1)

            @pl.when(c0 + 3 < n_full)
            def _():
                issue(pl.multiple_of(e0a + 3 * CHUNK, 128), buf1, sem1)

            return f

        f = lax.fori_loop(0, n_pairs, pbody, jnp.int32(0))

        def odd_fn(ff):
            drain(buf0, sem0)
            lo_loc = (n_full - 1) * CHUNK
            return chunk_process(ff, lo_loc, lo_loc + CHUNK, buf0)

        f = lax.cond((n_full & 1) == 1, odd_fn, lambda ff: ff, f)

        def tail_fn(ff):
            e0g = pl.multiple_of(e_lo + n_full * CHUNK, 128)
            cps = [
                pltpu.async_copy(
                    et_h.at[pl.ds(a * 8, 8), pl.ds(e0g, 64)],
                    buf1.at[pl.ds(a * 8, 8), pl.ds(0, 64)],
                    sem1,
                )
                for a in range(D // 8)
            ]
            for cp in cps:
                cp.wait()
            return chunk_process(ff, n_full * CHUNK, span, buf1)

        f = lax.cond(rem > 0, tail_fn, lambda ff: ff, f)
        lax.cond(f > 0, flush, lambda x: x, f)

    return ka(heads, tails, et)


def _score_rows(rels, times, rel2, time2, rows):
    rows_per_w = B // N_WORKERS
    n_sub = rows_per_w // 128
    mesh = plsc.VectorSubcoreMesh(core_axis_name="c", subcore_axis_name="s")

    @functools.partial(
        pl.kernel,
        mesh=mesh,
        compiler_params=_params,
        out_type=jax.ShapeDtypeStruct((B,), jnp.float32),
        scratch_types=[
            pltpu.VMEM((rows_per_w,), jnp.int32),
            pltpu.VMEM((rows_per_w,), jnp.int32),
            pltpu.VMEM((rows_per_w,), jnp.int32),
            pltpu.VMEM((rows_per_w,), jnp.int32),
            pltpu.VMEM((128, 2 * D), jnp.float32),
            pltpu.VMEM((128, 2 * D), jnp.float32),
            pltpu.VMEM((128, 2 * D), jnp.float32),
            pltpu.VMEM((128, 2 * D), jnp.float32),
            pltpu.VMEM((rows_per_w,), jnp.float32),
            pltpu.SemaphoreType.DMA,
        ],
    )
    def kb(rels_h, times_h, rel_h, time_h, rows_h, out_h,
           ridx, midx, rgi, mgi, hb, tb, rb, mb, ob, sem):
        w = lax.axis_index("s") * 2 + lax.axis_index("c")
        base = w * rows_per_w

        pltpu.sync_copy(rels_h.at[pl.ds(base, rows_per_w)], ridx)
        pltpu.sync_copy(times_h.at[pl.ds(base, rows_per_w)], midx)

        def sbody(i, _):
            sl = pl.ds(i * LANES, LANES)
            rgi[sl] = ridx[sl] >> 1
            mgi[sl] = midx[sl] >> 1
            return _

        lax.fori_loop(0, rows_per_w // LANES, sbody, jnp.int32(0))

        lane = lax.iota(jnp.int32, LANES)

        for j in range(n_sub):
            sl = pl.ds(j * 128, 128)
            c1 = pltpu.async_copy(rows_h.at[pl.ds(base + j * 128, 128)], hb, sem)
            c2 = pltpu.async_copy(
                rows_h.at[pl.ds(B + base + j * 128, 128)], tb, sem
            )
            c3 = pltpu.async_copy(rel_h.at[rgi.at[sl]], rb, sem)
            c4 = pltpu.async_copy(time_h.at[mgi.at[sl]], mb, sem)
            c1.wait()
            c2.wait()
            c3.wait()
            c4.wait()

            def gbody(g, _):
                gsl = pl.ds(j * 128 + g * LANES, LANES)
                roff = (ridx[gsl] & 1) * D
                moff = (midx[gsl] & 1) * D
                v = jnp.zeros((LANES,), jnp.float32)
                for r in range(LANES):
                    row = g * LANES + r
                    part = jnp.zeros((LANES,), jnp.float32)
                    for c in range(D // LANES):
                        s = (hb[row, pl.ds(c * LANES, LANES)]
                             + rb[row, pl.ds(roff[r] + c * LANES, LANES)]
                             + mb[row, pl.ds(moff[r] + c * LANES, LANES)]
                             - tb[row, pl.ds(c * LANES, LANES)])
                        part = part + s * s
                    v = jnp.where(lane == jnp.int32(r), jnp.sum(part), v)
                ob[gsl] = _neg_norm(v)
                return _

            lax.fori_loop(0, 128 // LANES, gbody, jnp.int32(0))

        pltpu.sync_copy(ob, out_h.at[pl.ds(base, rows_per_w)])

    return kb(rels, times, rel2, time2, rows)


def kernel(heads, rels, tails, times, entity_table, relation_table, time_table):
    n_ent = entity_table.shape[0]
    et = entity_table.T
    rel2 = jnp.reshape(relation_table, (-1, 2 * D))
    time2 = jnp.reshape(time_table, (-1, 2 * D))
    rows = _gather_rows(heads, tails, et, n_ent)
    return _score_rows(rels, times, rel2, time2, rows)


# speedup vs baseline: 2.8035x; 1.0061x over previous
import functools

import jax
import jax.numpy as jnp
from jax import lax
from jax.experimental import pallas as pl
from jax.experimental.pallas import tpu as pltpu
from jax.experimental.pallas import tpu_sc as plsc

D = 64
N_WORKERS = 32
LANES = 16
CHUNK = 512
LIST_CAP = 32784
MINI_CAP = 2064
GCAP = 640
B = 16384
N_ROWS = 2 * B + N_WORKERS

_params = pltpu.CompilerParams(
    needs_layout_passes=False, use_tc_tiling_on_sc=True
)


def _neg_norm(x):
    xs = jnp.maximum(x, jnp.float32(1e-30))
    i = lax.bitcast_convert_type(xs, jnp.int32)
    y = lax.bitcast_convert_type(jnp.int32(0x5F3759DF) - (i >> 1), jnp.float32)
    half = jnp.float32(0.5) * xs
    for _ in range(3):
        y = y * (jnp.float32(1.5) - half * y * y)
    return -(xs * y)


def _gather_rows(heads, tails, et, n_ent):
    mesh = plsc.VectorSubcoreMesh(core_axis_name="c", subcore_axis_name="s")

    @functools.partial(
        pl.kernel,
        mesh=mesh,
        compiler_params=_params,
        out_type=jax.ShapeDtypeStruct((N_ROWS, 2 * D), jnp.float32),
        scratch_types=[
            pltpu.VMEM((2048,), jnp.int32),
            pltpu.VMEM((LIST_CAP,), jnp.int32),
            pltpu.VMEM((8 * GCAP,), jnp.int32),
            pltpu.VMEM((D, CHUNK), jnp.float32),
            pltpu.VMEM((D, CHUNK), jnp.float32),
            pltpu.VMEM((MINI_CAP,), jnp.int32),
            pltpu.VMEM((128, 2 * D), jnp.float32),
            pltpu.VMEM((1, 128), jnp.int32),
            pltpu.SemaphoreType.DMA,
            pltpu.SemaphoreType.DMA,
            pltpu.SemaphoreType.DMA,
        ],
    )
    def ka(heads_h, tails_h, et_h, rows_h,
           istage, plist, glist, buf0, buf1, mini, rowbuf, idxst,
           sem0, sem1, semS):
        w = lax.axis_index("s") * 2 + lax.axis_index("c")
        lane = lax.iota(jnp.int32, LANES)
        dumpv = jnp.full((LANES,), jnp.int32(2 * B)) + w

        e_lo = w << 15
        e_hi = jnp.minimum((w + 1) << 15, n_ent)
        span = jnp.maximum(e_hi - e_lo, 0)
        n_full = span // CHUNK
        rem = span - n_full * CHUNK

        def issue(e0g, buf, sm):
            pltpu.async_copy(
                et_h.at[pl.ds(0, D), pl.ds(e0g, CHUNK)], buf, sm
            )

        @pl.when(n_full > 0)
        def _():
            issue(pl.multiple_of(e_lo, 128), buf0, sem0)

        @pl.when(n_full > 1)
        def _():
            issue(pl.multiple_of(e_lo + CHUNK, 128), buf1, sem1)

        n = jnp.int32(0)
        for src, rid_base in ((heads_h, 0), (tails_h, B)):
            for cb in range(B // 2048):
                pltpu.sync_copy(src.at[pl.ds(cb * 2048, 2048)], istage)

                def pbody(v4, n, _rb=rid_base + cb * 2048):
                    for u in range(4):
                        v = v4 * 4 + u
                        ev = istage[pl.ds(v * LANES, LANES)]
                        m = (ev >> 15) == w
                        rid = lane + (v * LANES + _rb)
                        pk = ((ev & 32767) << 15) | rid
                        plsc.store_compressed(
                            plist.at[pl.ds(n, LANES)], pk, mask=m
                        )
                        n = n + plsc.all_reduce_population_count(m)[0]
                    return n

                n = lax.fori_loop(0, 2048 // LANES // 4, pbody, n)

        sent = jnp.full((LANES,), jnp.int32(0x7FFFFFFF))
        plist[pl.ds(n, LANES)] = sent

        nv_all = (n + LANES - 1) >> 4

        def gbody(v, carry):
            pk = plist[pl.ds(v * LANES, LANES)]
            gid = pk >> 27
            out = []
            for g in range(8):
                m = gid == g
                pos = jnp.minimum(carry[g], GCAP - LANES)
                plsc.store_compressed(
                    glist.at[pl.ds(g * GCAP + pos, LANES)], pk, mask=m
                )
                out.append(carry[g] + plsc.all_reduce_population_count(m)[0])
            return tuple(out)

        gcnts = lax.fori_loop(
            0, nv_all, gbody, (jnp.int32(0),) * 8
        )
        gvec = jnp.zeros((LANES,), jnp.int32)
        for g in range(8):
            glist[pl.ds(g * GCAP + jnp.minimum(gcnts[g], GCAP - LANES), LANES)] = sent
            gvec = jnp.where(lane == g, gcnts[g], gvec)

        for kk in range(8):
            idxst[0, pl.ds(kk * LANES, LANES)] = dumpv

        def flush(_):
            pltpu.async_copy(rowbuf, rows_h.at[idxst.at[0]], semS).wait()
            for kk in range(8):
                idxst[0, pl.ds(kk * LANES, LANES)] = dumpv
            return jnp.int32(0)

        def process_mini(k, f, lo_loc, buf):
            def mbody(mv, f):
                f = lax.cond(f > jnp.int32(112), flush, lambda x: x, f)
                mp = mini[pl.ds(mv * LANES, LANES)]
                for r in range(LANES):
                    @pl.when(mv * LANES + r < k)
                    def _():
                        pk = mp[r]
                        col = jnp.zeros((LANES,), jnp.int32) + (
                            (pk >> 15) - lo_loc
                        )
                        fr = f + r
                        for c in range(D // LANES):
                            fv = plsc.load_gather(buf, [lane + c * LANES, col])
                            rowbuf[fr, pl.ds(c * LANES, LANES)] = fv
                        jrow = (fr >> 4) * LANES
                        cur = idxst[0, pl.ds(jrow, LANES)]
                        idxst[0, pl.ds(jrow, LANES)] = jnp.where(
                            lane == (fr & 15), pk & 32767, cur
                        )
                return f + jnp.minimum(k - mv * LANES, LANES)

            return lax.fori_loop(0, (k + LANES - 1) >> 4, mbody, f)

        def chunk_process(f, lo_loc, hi_loc, buf):
            lo = lo_loc << 15
            hi = hi_loc << 15
            g = lo_loc >> 12
            cnt_g = jnp.sum(jnp.where(lane == g, gvec, jnp.int32(0)))

            def scan_vreg(ref):
                def sbody(v, mcnt):
                    pk = ref[pl.ds(v * LANES, LANES)]
                    m = (pk >= lo) & (pk < hi)
                    plsc.store_compressed(
                        mini.at[pl.ds(mcnt, LANES)], pk, mask=m
                    )
                    return mcnt + plsc.all_reduce_population_count(m)[0]

                return sbody

            def group_scan(ff):
                v0 = g * (GCAP // LANES)
                mcnt = lax.fori_loop(
                    v0, v0 + ((cnt_g + LANES - 1) >> 4), scan_vreg(glist),
                    jnp.int32(0),
                )
                return (mcnt, ff)

            def full_scan(ff):
                nv = (n + LANES - 1) >> 4
                nb = (nv + 63) >> 6

                def bbody(bi, carry):
                    mcnt, ff = carry
                    mcnt, ff = lax.cond(
                        mcnt >= jnp.int32(MINI_CAP - 1040),
                        lambda c, f2: (
                            jnp.int32(0), process_mini(c, f2, lo_loc, buf)
                        ),
                        lambda c, f2: (c, f2),
                        mcnt, ff,
                    )
                    v0 = bi * 64
                    mcnt = lax.fori_loop(
                        v0, jnp.minimum(v0 + 64, nv), scan_vreg(plist), mcnt
                    )
                    return (mcnt, ff)

                return lax.fori_loop(0, nb, bbody, (jnp.int32(0), ff))

            mcnt, f = lax.cond(
                cnt_g > jnp.int32(GCAP - LANES), full_scan, group_scan, f
            )
            return process_mini(mcnt, f, lo_loc, buf)

        def drain(buf, sm):
            pltpu.make_async_copy(
                et_h.at[pl.ds(0, D), pl.ds(0, CHUNK)], buf, sm
            ).wait()

        n_pairs = n_full >> 1

        def pbody(ci2, f):
            c0 = ci2 * 2
            e0a = pl.multiple_of(e_lo + c0 * CHUNK, 128)
            lo_a = c0 * CHUNK
            drain(buf0, sem0)
            f = chunk_process(f, lo_a, lo_a + CHUNK, buf0)

            @pl.when(c0 + 2 < n_full)
            def _():
                issue(pl.multiple_of(e0a + 2 * CHUNK, 128), buf0, sem0)

            drain(buf1, sem1)
            f = chunk_process(f, lo_a + CHUNK, lo_a + 2 * CHUNK, buf1)

            @pl.when(c0 + 3 < n_full)
            def _():
                issue(pl.multiple_of(e0a + 3 * CHUNK, 128), buf1, sem1)

            return f

        f = lax.fori_loop(0, n_pairs, pbody, jnp.int32(0))

        def odd_fn(ff):
            drain(buf0, sem0)
            lo_loc = (n_full - 1) * CHUNK
            return chunk_process(ff, lo_loc, lo_loc + CHUNK, buf0)

        f = lax.cond((n_full & 1) == 1, odd_fn, lambda ff: ff, f)

        def tail_fn(ff):
            e0g = pl.multiple_of(e_lo + n_full * CHUNK, 128)
            cps = [
                pltpu.async_copy(
                    et_h.at[pl.ds(a * 8, 8), pl.ds(e0g, 64)],
                    buf1.at[pl.ds(a * 8, 8), pl.ds(0, 64)],
                    sem1,
                )
                for a in range(D // 8)
            ]
            for cp in cps:
                cp.wait()
            return chunk_process(ff, n_full * CHUNK, span, buf1)

        f = lax.cond(rem > 0, tail_fn, lambda ff: ff, f)
        lax.cond(f > 0, flush, lambda x: x, f)

    return ka(heads, tails, et)


def _score_rows(rels, times, rel2, time2, rows):
    rows_per_w = B // N_WORKERS
    SUBB = 64
    n_sub = rows_per_w // SUBB
    mesh = plsc.VectorSubcoreMesh(core_axis_name="c", subcore_axis_name="s")

    @functools.partial(
        pl.kernel,
        mesh=mesh,
        compiler_params=_params,
        out_type=jax.ShapeDtypeStruct((B,), jnp.float32),
        scratch_types=[
            pltpu.VMEM((rows_per_w,), jnp.int32),
            pltpu.VMEM((rows_per_w,), jnp.int32),
            pltpu.VMEM((rows_per_w,), jnp.int32),
            pltpu.VMEM((rows_per_w,), jnp.int32),
            pltpu.VMEM((SUBB, 2 * D), jnp.float32),
            pltpu.VMEM((SUBB, 2 * D), jnp.float32),
            pltpu.VMEM((SUBB, 2 * D), jnp.float32),
            pltpu.VMEM((SUBB, 2 * D), jnp.float32),
            pltpu.VMEM((SUBB, 2 * D), jnp.float32),
            pltpu.VMEM((SUBB, 2 * D), jnp.float32),
            pltpu.VMEM((SUBB, 2 * D), jnp.float32),
            pltpu.VMEM((SUBB, 2 * D), jnp.float32),
            pltpu.VMEM((rows_per_w,), jnp.float32),
            pltpu.SemaphoreType.DMA,
            pltpu.SemaphoreType.DMA,
        ],
    )
    def kb(rels_h, times_h, rel_h, time_h, rows_h, out_h,
           ridx, midx, rgi, mgi, hb0, tb0, rb0, mb0, hb1, tb1, rb1, mb1,
           ob, semA, semB):
        w = lax.axis_index("s") * 2 + lax.axis_index("c")
        base = w * rows_per_w

        pltpu.sync_copy(rels_h.at[pl.ds(base, rows_per_w)], ridx)
        pltpu.sync_copy(times_h.at[pl.ds(base, rows_per_w)], midx)

        def sbody(i, _):
            sl = pl.ds(i * LANES, LANES)
            rgi[sl] = ridx[sl] >> 1
            mgi[sl] = midx[sl] >> 1
            return _

        lax.fori_loop(0, rows_per_w // LANES, sbody, jnp.int32(0))

        lane = lax.iota(jnp.int32, LANES)
        bufs = ((hb0, tb0, rb0, mb0, semA), (hb1, tb1, rb1, mb1, semB))

        def issue(j, bs):
            hb, tb, rb, mb, sm = bs
            sl = pl.ds(j * SUBB, SUBB)
            pltpu.async_copy(rows_h.at[pl.ds(base + j * SUBB, SUBB)], hb, sm)
            pltpu.async_copy(
                rows_h.at[pl.ds(B + base + j * SUBB, SUBB)], tb, sm
            )
            pltpu.async_copy(rel_h.at[rgi.at[sl]], rb, sm)
            pltpu.async_copy(time_h.at[mgi.at[sl]], mb, sm)

        def drain(bs):
            hb, tb, rb, mb, sm = bs
            for dst in (hb, tb, rb, mb):
                pltpu.make_async_copy(
                    rows_h.at[pl.ds(0, SUBB)], dst, sm
                ).wait()

        def compute(j, bs):
            hb, tb, rb, mb, _ = bs

            def gbody(g, _):
                gsl = pl.ds(j * SUBB + g * LANES, LANES)
                roff = (ridx[gsl] & 1) * D
                moff = (midx[gsl] & 1) * D
                v = jnp.zeros((LANES,), jnp.float32)
                for r in range(LANES):
                    row = g * LANES + r
                    part = jnp.zeros((LANES,), jnp.float32)
                    for c in range(D // LANES):
                        s = (hb[row, pl.ds(c * LANES, LANES)]
                             + rb[row, pl.ds(roff[r] + c * LANES, LANES)]
                             + mb[row, pl.ds(moff[r] + c * LANES, LANES)]
                             - tb[row, pl.ds(c * LANES, LANES)])
                        part = part + s * s
                    v = jnp.where(lane == jnp.int32(r), jnp.sum(part), v)
                ob[gsl] = _neg_norm(v)
                return _

            lax.fori_loop(0, SUBB // LANES, gbody, jnp.int32(0))

        issue(0, bufs[0])
        issue(1, bufs[1])
        for jp in range(n_sub // 2):
            drain(bufs[0])
            compute(2 * jp, bufs[0])
            if 2 * jp + 2 < n_sub:
                issue(2 * jp + 2, bufs[0])
            drain(bufs[1])
            compute(2 * jp + 1, bufs[1])
            if 2 * jp + 3 < n_sub:
                issue(2 * jp + 3, bufs[1])

        pltpu.sync_copy(ob, out_h.at[pl.ds(base, rows_per_w)])

    return kb(rels, times, rel2, time2, rows)


def kernel(heads, rels, tails, times, entity_table, relation_table, time_table):
    n_ent = entity_table.shape[0]
    et = entity_table.T
    rel2 = jnp.reshape(relation_table, (-1, 2 * D))
    time2 = jnp.reshape(time_table, (-1, 2 * D))
    rows = _gather_rows(heads, tails, et, n_ent)
    return _score_rows(rels, times, rel2, time2, rows)
